# baseline (speedup 1.0000x reference)
import functools

import jax
import jax.numpy as jnp
from jax import lax
from jax.experimental import pallas as pl
from jax.experimental.pallas import tpu as pltpu
from jax.experimental.pallas import tpu_sc as plsc

_B = 16384
_D = 64

_info = plsc.get_sparse_core_info()
_NC = _info.num_cores
_NS = _info.num_subcores
_NW = _NC * _NS
_B_PER_W = _B // _NW
_K = 16
_NCH = _B_PER_W // _K

_mesh = plsc.VectorSubcoreMesh(core_axis_name="c", subcore_axis_name="s")


@functools.partial(
    pl.kernel,
    mesh=_mesh,
    out_type=jax.ShapeDtypeStruct((_B, _D), jnp.float32),
    scratch_types=[
        pltpu.VMEM((_B_PER_W,), jnp.int32),
        pltpu.SemaphoreType.DMA,
    ],
    compiler_params=pltpu.CompilerParams(use_tc_tiling_on_sc=True),
)
def _sc_gather(table_hbm, idx_hbm, out_hbm, idx_s, sem):
    wid = lax.axis_index("s") * _NC + lax.axis_index("c")
    base = wid * _B_PER_W
    pltpu.sync_copy(idx_hbm.at[pl.ds(base, _B_PER_W)], idx_s)

    def fire(j0):
        v = idx_s[pl.ds(j0, _K)]
        for t in range(_K):
            pltpu.async_copy(table_hbm.at[v[t]], out_hbm.at[base + j0 + t], sem)

    def drain(j0):
        for t in range(_K):
            pltpu.make_async_copy(
                table_hbm.at[0], out_hbm.at[base + j0 + t], sem
            ).wait()

    fire(0)

    def body(j, carry):
        fire(j * _K)
        drain((j - 1) * _K)
        return carry

    lax.fori_loop(1, _NCH, body, 0)
    drain((_NCH - 1) * _K)


def kernel(hyperparameters, embedding_table):
    idx = jnp.squeeze(hyperparameters, axis=1)
    return _sc_gather(embedding_table, idx)


# speedup vs baseline: 2.8203x; 2.8203x over previous
import functools

import jax
import jax.numpy as jnp
from jax import lax
from jax.experimental import pallas as pl
from jax.experimental.pallas import tpu as pltpu
from jax.experimental.pallas import tpu_sc as plsc

_B = 16384
_D = 64

_info = plsc.get_sparse_core_info()
_NC = _info.num_cores
_NS = _info.num_subcores
_NW = _NC * _NS
_B_PER_W = _B // _NW
_L = 16

_mesh = plsc.VectorSubcoreMesh(core_axis_name="c", subcore_axis_name="s")


@functools.partial(
    pl.kernel,
    mesh=_mesh,
    out_type=jax.ShapeDtypeStruct((_D, _B), jnp.float32),
    scratch_types=[
        pltpu.VMEM((_B_PER_W,), jnp.int32),
        pltpu.VMEM((_B_PER_W,), jnp.int32),
        pltpu.VMEM((_B_PER_W, 2 * _D), jnp.float32),
        pltpu.VMEM((_D, _B_PER_W), jnp.float32),
        pltpu.SemaphoreType.DMA,
    ],
    compiler_params=pltpu.CompilerParams(use_tc_tiling_on_sc=True, needs_layout_passes=False),
)
def _sc_gather(table_hbm, idx_hbm, out_hbm, idx_v, tidx_v, rows_v, outt_v, sem):
    wid = lax.axis_index("s") * _NC + lax.axis_index("c")
    base = wid * _B_PER_W
    pltpu.sync_copy(idx_hbm.at[pl.ds(base, _B_PER_W)], idx_v)

    def halve(j, carry):
        v = idx_v[pl.ds(j * _L, _L)]
        tidx_v[pl.ds(j * _L, _L)] = v >> 1
        return carry

    lax.fori_loop(0, _B_PER_W // _L, halve, 0)

    pltpu.async_copy(table_hbm.at[tidx_v], rows_v, sem).wait()

    lanes = lax.iota(jnp.int32, _L)

    def select(j, carry):
        r_vec = j * _L + lanes
        par = idx_v[pl.ds(j * _L, _L)] & 1
        c0 = par * _D
        for d in range(_D):
            outt_v[d, pl.ds(j * _L, _L)] = plsc.load_gather(
                rows_v, [r_vec, c0 + d]
            )
        return carry

    lax.fori_loop(0, _B_PER_W // _L, select, 0)

    pltpu.sync_copy(outt_v, out_hbm.at[:, pl.ds(base, _B_PER_W)])


def kernel(hyperparameters, embedding_table):
    idx = jnp.squeeze(hyperparameters, axis=1)
    table2 = jnp.reshape(embedding_table, (50000, 2 * _D))
    return _sc_gather(table2, idx).T
